# baseline (device time: 97256 ns/iter reference)
import jax
import jax.numpy as jnp
from jax import lax
from jax.experimental import pallas as pl
from jax.experimental.pallas import tpu as pltpu

N_DEV = 16
NA = 9
NB = 8


def kernel(x, Wq, K_ext, V_ext, Wo):
    B, Sq, Din = x.shape
    _, HD = Wq.shape
    Bg, Skv, Hq, Dh = K_ext.shape
    Hloc = HD // Dh
    Dout = Wo.shape[1]
    BSq = B * Sq
    bf16 = jnp.bfloat16

    pos = lax.axis_index("i")
    k2 = lax.dynamic_slice_in_dim(K_ext, pos * B, B, axis=0).astype(
        bf16).reshape(B * Skv, Hq * Dh)
    v2 = lax.dynamic_slice_in_dim(V_ext, pos * B, B, axis=0).astype(
        bf16).reshape(B * Skv, Hq * Dh)

    def body(x_ref, wq_ref, k_ref, v_ref, wo_ref, out_ref,
             wq_full, wo_full, ctx_full, ctx_s,
             aq_s, aq_r, ao_s, ao_r, bq_s, bq_r, bo_s, bo_r):
        my = lax.axis_index("i")
        left = lax.rem(my + N_DEV - 1, N_DEV)
        right = lax.rem(my + 1, N_DEV)

        wq_full[:, pl.ds(my * HD, HD)] = wq_ref[...].astype(bf16)
        wo_full[pl.ds(my * HD, HD), :] = wo_ref[...].astype(bf16)

        bar = pltpu.get_barrier_semaphore()
        for nbr in (left, right):
            pl.semaphore_signal(
                bar, inc=1, device_id=(nbr,),
                device_id_type=pl.DeviceIdType.MESH)
        pl.semaphore_wait(bar, 2)

        x2 = x_ref[...].reshape(BSq, Din).astype(bf16)

        def ja(h):
            return lax.rem(my - h + N_DEV, N_DEV)

        def jb(h):
            return lax.rem(my + h, N_DEV)

        HH = Din // 2

        def mk_aq(h, c):
            sl = pl.ds(ja(h) * HD, HD)
            rw = pl.ds(c * HH, HH)
            return pltpu.make_async_remote_copy(
                src_ref=wq_full.at[rw, sl], dst_ref=wq_full.at[rw, sl],
                send_sem=aq_s.at[h, c], recv_sem=aq_r.at[h, c],
                device_id=(right,), device_id_type=pl.DeviceIdType.MESH)

        def mk_ao(h, c):
            sl = pl.ds(ja(h) * HD, HD)
            cw = pl.ds(c * (Dout // 2), Dout // 2)
            return pltpu.make_async_remote_copy(
                src_ref=wo_full.at[sl, cw], dst_ref=wo_full.at[sl, cw],
                send_sem=ao_s.at[h, c], recv_sem=ao_r.at[h, c],
                device_id=(right,), device_id_type=pl.DeviceIdType.MESH)

        def mk_bq(h, c):
            sl = pl.ds(jb(h) * HD, HD)
            rw = pl.ds(c * HH, HH)
            return pltpu.make_async_remote_copy(
                src_ref=wq_full.at[rw, sl], dst_ref=wq_full.at[rw, sl],
                send_sem=bq_s.at[h, c], recv_sem=bq_r.at[h, c],
                device_id=(left,), device_id_type=pl.DeviceIdType.MESH)

        def mk_bo(h, c):
            sl = pl.ds(jb(h) * HD, HD)
            cw = pl.ds(c * (Dout // 2), Dout // 2)
            return pltpu.make_async_remote_copy(
                src_ref=wo_full.at[sl, cw], dst_ref=wo_full.at[sl, cw],
                send_sem=bo_s.at[h, c], recv_sem=bo_r.at[h, c],
                device_id=(left,), device_id_type=pl.DeviceIdType.MESH)

        def compute(jj):
            sl = pl.ds(jj * HD, HD)
            q = jnp.dot(x2, wq_full[:, sl],
                        preferred_element_type=jnp.float32)
            for b in range(B):
                kb = k_ref[b * Skv:(b + 1) * Skv, sl]
                vb = v_ref[b * Skv:(b + 1) * Skv, sl]
                qb = q[b * Sq:(b + 1) * Sq, :].astype(bf16)
                for hh in range(Hloc):
                    qh = qb[:, hh * Dh:(hh + 1) * Dh]
                    kh = kb[:, hh * Dh:(hh + 1) * Dh]
                    vh = vb[:, hh * Dh:(hh + 1) * Dh]
                    s = lax.dot_general(
                        qh, kh, (((1,), (1,)), ((), ())),
                        preferred_element_type=jnp.float32)
                    w = jnp.exp(s * 0.125)
                    w = (w / jnp.sum(w, axis=-1, keepdims=True)).astype(bf16)
                    ctx_s[b * Sq:(b + 1) * Sq, hh * Dh:(hh + 1) * Dh] = (
                        jnp.dot(w, vh,
                                preferred_element_type=jnp.float32)
                        .astype(bf16))
            ctx_full[:, sl] = ctx_s[...]

        for c in range(2):
            mk_aq(0, c).start()
            mk_ao(0, c).start()
            mk_bq(0, c).start()
            mk_bo(0, c).start()
        compute(my)

        def hop(h, carry):
            for c in range(2):
                mk_aq(h - 1, c).wait_recv()
                mk_aq(h, c).start()
            for c in range(2):
                mk_ao(h - 1, c).wait_recv()
                mk_ao(h, c).start()

            @pl.when(h < NB - 1)
            def _():
                for c in range(2):
                    mk_bq(h - 1, c).wait_recv()
                    mk_bq(h, c).start()
                for c in range(2):
                    mk_bo(h - 1, c).wait_recv()
                    mk_bo(h, c).start()

            compute(ja(h))

            @pl.when(h < NB - 1)
            def _():
                compute(jb(h))
            return carry

        lax.fori_loop(1, NA - 1, hop, None)

        for c in range(2):
            mk_bq(NB - 2, c).wait_recv()
            mk_bo(NB - 2, c).wait_recv()
        compute(jb(NB - 1))
        for c in range(2):
            mk_aq(NA - 2, c).wait_recv()
            mk_ao(NA - 2, c).wait_recv()
        compute(ja(NA - 1))

        for h in range(NA - 1):
            for c in range(2):
                mk_aq(h, c).wait_send()
                mk_ao(h, c).wait_send()
        for h in range(NB - 1):
            for c in range(2):
                mk_bq(h, c).wait_send()
                mk_bo(h, c).wait_send()

        out = jnp.dot(ctx_full[...], wo_full[...],
                      preferred_element_type=jnp.float32)
        out_ref[...] = out.reshape(B, Sq, Dout)

    return pl.pallas_call(
        body,
        out_shape=jax.ShapeDtypeStruct((B, Sq, Dout), jnp.float32),
        in_specs=[
            pl.BlockSpec(memory_space=pltpu.VMEM),
            pl.BlockSpec(memory_space=pltpu.VMEM),
            pl.BlockSpec(memory_space=pltpu.VMEM),
            pl.BlockSpec(memory_space=pltpu.VMEM),
            pl.BlockSpec(memory_space=pltpu.VMEM),
        ],
        out_specs=pl.BlockSpec(memory_space=pltpu.VMEM),
        scratch_shapes=[
            pltpu.VMEM((Din, N_DEV * HD), bf16),
            pltpu.VMEM((N_DEV * HD, Dout), bf16),
            pltpu.VMEM((BSq, N_DEV * HD), bf16),
            pltpu.VMEM((BSq, HD), bf16),
            pltpu.SemaphoreType.DMA((NA - 1, 2)),
            pltpu.SemaphoreType.DMA((NA - 1, 2)),
            pltpu.SemaphoreType.DMA((NA - 1, 2)),
            pltpu.SemaphoreType.DMA((NA - 1, 2)),
            pltpu.SemaphoreType.DMA((NB - 1, 2)),
            pltpu.SemaphoreType.DMA((NB - 1, 2)),
            pltpu.SemaphoreType.DMA((NB - 1, 2)),
            pltpu.SemaphoreType.DMA((NB - 1, 2)),
        ],
        compiler_params=pltpu.CompilerParams(
            collective_id=0, vmem_limit_bytes=56 * 1024 * 1024),
    )(x, Wq, k2, v2, Wo)


# device time: 96607 ns/iter; 1.0067x vs baseline; 1.0067x over previous
import jax
import jax.numpy as jnp
from jax import lax
from jax.experimental import pallas as pl
from jax.experimental.pallas import tpu as pltpu

N_DEV = 16
NA = 9
NB = 8


def kernel(x, Wq, K_ext, V_ext, Wo):
    B, Sq, Din = x.shape
    _, HD = Wq.shape
    Bg, Skv, Hq, Dh = K_ext.shape
    Hloc = HD // Dh
    Dout = Wo.shape[1]
    BSq = B * Sq
    bf16 = jnp.bfloat16

    pos = lax.axis_index("i")
    k2 = lax.dynamic_slice_in_dim(K_ext, pos * B, B, axis=0).astype(
        bf16).reshape(B * Skv, Hq * Dh)
    v2 = lax.dynamic_slice_in_dim(V_ext, pos * B, B, axis=0).astype(
        bf16).reshape(B * Skv, Hq * Dh)

    def body(x_ref, wq_ref, k_ref, v_ref, wo_ref, out_ref,
             wq_full, wo_full, ctx_full, ctx_s,
             aq_s, aq_r, ao_s, ao_r, bq_s, bq_r, bo_s, bo_r):
        my = lax.axis_index("i")
        left = lax.rem(my + N_DEV - 1, N_DEV)
        right = lax.rem(my + 1, N_DEV)

        wq_full[:, pl.ds(my * HD, HD)] = wq_ref[...].astype(bf16)
        wo_full[pl.ds(my * HD, HD), :] = wo_ref[...].astype(bf16)

        bar = pltpu.get_barrier_semaphore()
        for nbr in (left, right):
            pl.semaphore_signal(
                bar, inc=1, device_id=(nbr,),
                device_id_type=pl.DeviceIdType.MESH)
        pl.semaphore_wait(bar, 2)

        x2 = x_ref[...].reshape(BSq, Din).astype(bf16)

        def ja(h):
            return lax.rem(my - h + N_DEV, N_DEV)

        def jb(h):
            return lax.rem(my + h, N_DEV)

        def mk_aq(h):
            sl = pl.ds(ja(h) * HD, HD)
            return pltpu.make_async_remote_copy(
                src_ref=wq_full.at[:, sl], dst_ref=wq_full.at[:, sl],
                send_sem=aq_s.at[h], recv_sem=aq_r.at[h],
                device_id=(right,), device_id_type=pl.DeviceIdType.MESH)

        def mk_ao(h):
            sl = pl.ds(ja(h) * HD, HD)
            return pltpu.make_async_remote_copy(
                src_ref=wo_full.at[sl, :], dst_ref=wo_full.at[sl, :],
                send_sem=ao_s.at[h], recv_sem=ao_r.at[h],
                device_id=(right,), device_id_type=pl.DeviceIdType.MESH)

        def mk_bq(h):
            sl = pl.ds(jb(h) * HD, HD)
            return pltpu.make_async_remote_copy(
                src_ref=wq_full.at[:, sl], dst_ref=wq_full.at[:, sl],
                send_sem=bq_s.at[h], recv_sem=bq_r.at[h],
                device_id=(left,), device_id_type=pl.DeviceIdType.MESH)

        def mk_bo(h):
            sl = pl.ds(jb(h) * HD, HD)
            return pltpu.make_async_remote_copy(
                src_ref=wo_full.at[sl, :], dst_ref=wo_full.at[sl, :],
                send_sem=bo_s.at[h], recv_sem=bo_r.at[h],
                device_id=(left,), device_id_type=pl.DeviceIdType.MESH)

        def compute(jj):
            sl = pl.ds(jj * HD, HD)
            q = jnp.dot(x2, wq_full[:, sl],
                        preferred_element_type=jnp.float32)
            for b in range(B):
                kb = k_ref[b * Skv:(b + 1) * Skv, sl]
                vb = v_ref[b * Skv:(b + 1) * Skv, sl]
                qb = q[b * Sq:(b + 1) * Sq, :].astype(bf16)
                for hh in range(Hloc):
                    qh = qb[:, hh * Dh:(hh + 1) * Dh]
                    kh = kb[:, hh * Dh:(hh + 1) * Dh]
                    vh = vb[:, hh * Dh:(hh + 1) * Dh]
                    s = lax.dot_general(
                        qh, kh, (((1,), (1,)), ((), ())),
                        preferred_element_type=jnp.float32)
                    w = jnp.exp(s * 0.125)
                    w = (w / jnp.sum(w, axis=-1, keepdims=True)).astype(bf16)
                    ctx_s[b * Sq:(b + 1) * Sq, hh * Dh:(hh + 1) * Dh] = (
                        jnp.dot(w, vh,
                                preferred_element_type=jnp.float32)
                        .astype(bf16))
            ctx_full[:, sl] = ctx_s[...]

        mk_aq(0).start()
        mk_ao(0).start()
        mk_bq(0).start()
        mk_bo(0).start()
        compute(my)

        def hop(h, carry):
            mk_aq(h - 1).wait_recv()
            mk_aq(h).start()
            mk_ao(h - 1).wait_recv()
            mk_ao(h).start()

            @pl.when(h < NB - 1)
            def _():
                mk_bq(h - 1).wait_recv()
                mk_bq(h).start()
                mk_bo(h - 1).wait_recv()
                mk_bo(h).start()

            compute(ja(h))

            @pl.when(h < NB - 1)
            def _():
                compute(jb(h))
            return carry

        lax.fori_loop(1, NA - 1, hop, None)

        mk_bq(NB - 2).wait_recv()
        mk_bo(NB - 2).wait_recv()
        compute(jb(NB - 1))
        mk_aq(NA - 2).wait_recv()
        mk_ao(NA - 2).wait_recv()
        compute(ja(NA - 1))

        for h in range(NA - 1):
            mk_aq(h).wait_send()
            mk_ao(h).wait_send()
        for h in range(NB - 1):
            mk_bq(h).wait_send()
            mk_bo(h).wait_send()

        out = jnp.dot(ctx_full[...], wo_full[...],
                      preferred_element_type=jnp.float32)
        out_ref[...] = out.reshape(B, Sq, Dout)

    return pl.pallas_call(
        body,
        out_shape=jax.ShapeDtypeStruct((B, Sq, Dout), jnp.float32),
        in_specs=[
            pl.BlockSpec(memory_space=pltpu.VMEM),
            pl.BlockSpec(memory_space=pltpu.VMEM),
            pl.BlockSpec(memory_space=pltpu.VMEM),
            pl.BlockSpec(memory_space=pltpu.VMEM),
            pl.BlockSpec(memory_space=pltpu.VMEM),
        ],
        out_specs=pl.BlockSpec(memory_space=pltpu.VMEM),
        scratch_shapes=[
            pltpu.VMEM((Din, N_DEV * HD), bf16),
            pltpu.VMEM((N_DEV * HD, Dout), bf16),
            pltpu.VMEM((BSq, N_DEV * HD), bf16),
            pltpu.VMEM((BSq, HD), bf16),
            pltpu.SemaphoreType.DMA((NA - 1,)),
            pltpu.SemaphoreType.DMA((NA - 1,)),
            pltpu.SemaphoreType.DMA((NA - 1,)),
            pltpu.SemaphoreType.DMA((NA - 1,)),
            pltpu.SemaphoreType.DMA((NB - 1,)),
            pltpu.SemaphoreType.DMA((NB - 1,)),
            pltpu.SemaphoreType.DMA((NB - 1,)),
            pltpu.SemaphoreType.DMA((NB - 1,)),
        ],
        compiler_params=pltpu.CompilerParams(
            collective_id=0, vmem_limit_bytes=56 * 1024 * 1024),
    )(x, Wq, k2, v2, Wo)
